# baseline (speedup 1.0000x reference)
import functools
import jax
import jax.numpy as jnp
from jax import lax
from jax.experimental import pallas as pl
from jax.experimental.pallas import tpu as pltpu
from jax.experimental.pallas import tpu_sc as plsc

_TOPK = 2
_BT = 64
_NJ = 4
_BTA = 512
_NW = 32
_CH = 64

def _sc_mesh():
    return plsc.VectorSubcoreMesh(core_axis_name="c", subcore_axis_name="s",
                                  num_cores=2, num_subcores=16)


def _router_body(x_ref, rw_ref, eidx_ref, wts_ref, cnt_ref):
    E = rw_ref.shape[0]
    l = jax.lax.dot_general(rw_ref[...], x_ref[...], (((1,), (1,)), ((), ())),
                            preferred_element_type=jnp.float32)
    ioe = lax.broadcasted_iota(jnp.int32, l.shape, 0)
    m1 = jnp.max(l, axis=0)
    i1 = jnp.min(jnp.where(l == m1[None, :], ioe, E), axis=0)
    lm = jnp.where(ioe == i1[None, :], jnp.float32(-1e30), l)
    m2 = jnp.max(lm, axis=0)
    i2 = jnp.min(jnp.where(lm == m2[None, :], ioe, E), axis=0)
    w1 = 1.0 / (1.0 + jnp.exp(m2 - m1))
    eidx_ref[0, :] = i1
    eidx_ref[1, :] = i2
    wts_ref[0, :] = w1
    wts_ref[1, :] = 1.0 - w1
    oh = ((ioe == i1[None, :]).astype(jnp.int32)
          + (ioe == i2[None, :]).astype(jnp.int32))
    for g in range(_BTA // _CH):
        cnt_ref[g, :] = jnp.sum(oh[:, g * _CH:(g + 1) * _CH], axis=1)


def _router(x, router_w):
    T, H = x.shape
    E = router_w.shape[0]
    nb = T // _BTA
    return pl.pallas_call(
        _router_body,
        grid=(nb,),
        in_specs=[
            pl.BlockSpec((_BTA, H), lambda i: (i, 0)),
            pl.BlockSpec((E, H), lambda i: (0, 0)),
        ],
        out_specs=[
            pl.BlockSpec((_TOPK, _BTA), lambda i: (0, i)),
            pl.BlockSpec((_TOPK, _BTA), lambda i: (0, i)),
            pl.BlockSpec((_BTA // _CH, E), lambda i: (i, 0)),
        ],
        out_shape=[
            jax.ShapeDtypeStruct((_TOPK, T), jnp.int32),
            jax.ShapeDtypeStruct((_TOPK, T), jnp.float32),
            jax.ShapeDtypeStruct((T // _CH, E), jnp.int32),
        ],
    )(x, router_w)


def _pos_body(eidx_ref, cnt_ref, pos_ref, te_ref, carry_ref, *, E, NB, NT):
    B = eidx_ref.shape[1]
    i = pl.program_id(0)

    @pl.when(i == 0)
    def _():
        carry_ref[...] = jnp.zeros_like(carry_ref)

    e1 = eidx_ref[0, :]
    e2 = eidx_ref[1, :]
    ioe = lax.broadcasted_iota(jnp.int32, (E, B), 0)
    oh1 = (ioe == e1[None, :]).astype(jnp.float32)
    oh2 = (ioe == e2[None, :]).astype(jnp.float32)
    ohm = oh1 + oh2

    U = (lax.broadcasted_iota(jnp.int32, (B, B), 0)
         < lax.broadcasted_iota(jnp.int32, (B, B), 1)).astype(jnp.float32)
    carry = carry_ref[:, 0:1]
    prefix = jax.lax.dot_general(ohm, U, (((1,), (0,)), ((), ())),
                                 preferred_element_type=jnp.float32) + carry

    g = jnp.sum(cnt_ref[...].astype(jnp.float32), axis=0)
    gi = g.astype(jnp.int32)
    aligned = (((gi + (_BT - 1)) >> 6) << 6).astype(jnp.float32)
    UE = (lax.broadcasted_iota(jnp.int32, (E, E), 0)
          < lax.broadcasted_iota(jnp.int32, (E, E), 1)).astype(jnp.float32)
    excl = jax.lax.dot_general(aligned[None, :], UE, (((1,), (0,)), ((), ())),
                               preferred_element_type=jnp.float32)
    excl_col = excl.reshape(E, 1)

    posmat = prefix + excl_col
    pos_ref[0, :] = jnp.sum(oh1 * posmat, axis=0).astype(jnp.int32)
    pos_ref[1, :] = jnp.sum(oh2 * posmat, axis=0).astype(jnp.int32)
    carry_ref[:, 0:1] = carry + jnp.sum(ohm, axis=1, keepdims=True)

    @pl.when(i == NB - 1)
    def _():
        tstart = excl_col.astype(jnp.int32) >> 6
        iot = lax.broadcasted_iota(jnp.int32, (E, NT), 1)
        tev = jnp.sum((tstart <= iot).astype(jnp.int32), axis=0) - 1
        te_ref[0, :] = tev


def _positions(eidx, cnts, NT):
    K, T = eidx.shape
    E = cnts.shape[1]
    NB = T // _BTA
    return pl.pallas_call(
        functools.partial(_pos_body, E=E, NB=NB, NT=NT),
        grid=(NB,),
        in_specs=[
            pl.BlockSpec((K, _BTA), lambda i: (0, i)),
            pl.BlockSpec(cnts.shape, lambda i: (0, 0)),
        ],
        out_specs=[
            pl.BlockSpec((K, _BTA), lambda i: (0, i)),
            pl.BlockSpec((1, NT), lambda i: (0, 0)),
        ],
        out_shape=[
            jax.ShapeDtypeStruct((K, T), jnp.int32),
            jax.ShapeDtypeStruct((1, NT), jnp.int32),
        ],
        scratch_shapes=[pltpu.VMEM((E, 128), jnp.float32)],
    )(eidx, cnts)


def _dispatch(x, pos, NTOT):
    T, H = x.shape

    @functools.partial(
        pl.kernel,
        out_type=jax.ShapeDtypeStruct((NTOT, H), jnp.float32),
        mesh=_sc_mesh(),
        scratch_types=[
            pltpu.VMEM((_CH,), jnp.int32),
            pltpu.VMEM((_CH,), jnp.int32),
            pltpu.VMEM((_CH, H), jnp.float32),
            pltpu.SemaphoreType.DMA,
        ],
    )
    def body(x_hbm, pos_hbm, xs_hbm, ib1, ib2, xbuf, sem):
        wid = lax.axis_index("s") * 2 + lax.axis_index("c")
        base = wid * _CH
        pltpu.sync_copy(pos_hbm.at[0, pl.ds(base, _CH)], ib1)
        pltpu.sync_copy(pos_hbm.at[1, pl.ds(base, _CH)], ib2)
        pltpu.sync_copy(x_hbm.at[pl.ds(base, _CH), :], xbuf)
        pltpu.async_copy(xbuf, xs_hbm.at[ib1], sem).wait()
        pltpu.async_copy(xbuf, xs_hbm.at[ib2], sem).wait()

    return body(x, pos)


def _c_body(te_ref, x_ref, wgx_ref, wgg_ref, wu_ref, wd_ref, y_ref, acc_ref, xc_ref):
    j = pl.program_id(0)
    i = pl.program_id(1)
    rows = pl.ds(i * _BT, _BT)
    x = jnp.where(j == 0, x_ref[...], xc_ref[rows, :])

    @pl.when(j == 0)
    def _():
        xc_ref[rows, :] = x

    xh = jax.lax.dot_general(x, wgx_ref[0, 0], (((1,), (1,)), ((), ())),
                             preferred_element_type=jnp.float32)
    gt = jax.lax.dot_general(x, wgg_ref[0, 0], (((1,), (1,)), ((), ())),
                             preferred_element_type=jnp.float32)
    up = jax.lax.dot_general(x, wu_ref[0, 0], (((1,), (1,)), ((), ())),
                             preferred_element_type=jnp.float32)
    a = (gt * jax.nn.sigmoid(gt)) * xh * up
    part = jax.lax.dot_general(a, wd_ref[0], (((1,), (1,)), ((), ())),
                               preferred_element_type=jnp.float32)
    part = jnp.where(j == 0, part, acc_ref[rows, :] + part)
    acc_ref[rows, :] = part
    y_ref[...] = part


def _grouped_mlp(x_sorted, tile_expert, Wg, Wu, Wd):
    E, I2, H = Wg.shape
    I = I2 // 2
    NTOT = x_sorted.shape[0]
    NT = NTOT // _BT
    BI = I // _NJ
    WgB = Wg.reshape(E, 2 * _NJ, BI, H)

    y = pl.pallas_call(
        _c_body,
        grid_spec=pltpu.PrefetchScalarGridSpec(
            num_scalar_prefetch=1,
            grid=(_NJ, NT),
            in_specs=[
                pl.BlockSpec((_BT, H), lambda j, i, te: (jnp.where(j == 0, i, 0), 0)),
                pl.BlockSpec((1, 1, BI, H), lambda j, i, te: (te[i], j, 0, 0)),
                pl.BlockSpec((1, 1, BI, H), lambda j, i, te: (te[i], _NJ + j, 0, 0)),
                pl.BlockSpec((1, 1, BI, H), lambda j, i, te: (te[i], j, 0, 0)),
                pl.BlockSpec((1, H, BI), lambda j, i, te: (te[i], 0, j)),
            ],
            out_specs=pl.BlockSpec(
                (_BT, H), lambda j, i, te: (jnp.where(j == _NJ - 1, i, NT), 0)),
            scratch_shapes=[pltpu.VMEM((NTOT, H), jnp.float32),
                            pltpu.VMEM((NTOT, H), jnp.float32)],
        ),
        out_shape=jax.ShapeDtypeStruct(((NT + 1) * _BT, H), jnp.float32),
    )(tile_expert, x_sorted, WgB, WgB, Wu.reshape(E, _NJ, BI, H), Wd)
    return y


def _combine(y, pos, wts):
    NTOT, H = y.shape
    T = pos.shape[1]
    SUB = 32
    NCH = _CH // SUB

    @functools.partial(
        pl.kernel,
        out_type=jax.ShapeDtypeStruct((T, H), jnp.float32),
        mesh=_sc_mesh(),
        scratch_types=[
            pltpu.VMEM((SUB,), jnp.int32),
            pltpu.VMEM((SUB,), jnp.int32),
            pltpu.VMEM((SUB,), jnp.float32),
            pltpu.VMEM((SUB,), jnp.float32),
            pltpu.VMEM((SUB, H), jnp.float32),
            pltpu.VMEM((SUB, H), jnp.float32),
            pltpu.VMEM((SUB, H), jnp.float32),
            pltpu.SemaphoreType.DMA,
        ],
    )
    def body(y_hbm, pos_hbm, wts_hbm, out_hbm,
             p1b, p2b, w1b, w2b, ya, yb, ob, sem):
        wid = lax.axis_index("s") * 2 + lax.axis_index("c")
        for ch in range(NCH):
            cbase = wid * _CH + ch * SUB
            pltpu.sync_copy(pos_hbm.at[0, pl.ds(cbase, SUB)], p1b)
            pltpu.sync_copy(pos_hbm.at[1, pl.ds(cbase, SUB)], p2b)
            pltpu.sync_copy(wts_hbm.at[0, pl.ds(cbase, SUB)], w1b)
            pltpu.sync_copy(wts_hbm.at[1, pl.ds(cbase, SUB)], w2b)
            pltpu.async_copy(y_hbm.at[p1b], ya, sem).wait()
            pltpu.async_copy(y_hbm.at[p2b], yb, sem).wait()
            for cv in range(SUB // 16):
                w1v = w1b[pl.ds(cv * 16, 16)]
                w2v = w2b[pl.ds(cv * 16, 16)]
                for tt in range(16):
                    t = cv * 16 + tt
                    w1 = w1v[tt]
                    w2 = w2v[tt]
                    def cloop(c, _, t=t, w1=w1, w2=w2):
                        s = pl.ds(c * 16, 16)
                        ob[t, s] = w1 * ya[t, s] + w2 * yb[t, s]
                        return 0
                    lax.fori_loop(0, H // 16, cloop, 0)
            pltpu.sync_copy(ob, out_hbm.at[pl.ds(cbase, SUB), :])

    return body(y, pos, wts)


def kernel(hidden_states, router_w, Wg, Wu, Wd):
    B, S, H = hidden_states.shape
    E = router_w.shape[0]
    T = B * S
    x = hidden_states.reshape(T, H)
    NTOT = T * _TOPK + E * _BT
    NT = NTOT // _BT

    eidx, wts, cnts = _router(x, router_w)
    pos, te = _positions(eidx, cnts, NT)
    xs = _dispatch(x, pos, NTOT)
    y = _grouped_mlp(xs, te.reshape(NT), Wg, Wu, Wd)
    out = _combine(y, pos, wts)
    return out.reshape(B, S, H)


# speedup vs baseline: 1.1361x; 1.1361x over previous
import functools
import jax
import jax.numpy as jnp
from jax import lax
from jax.experimental import pallas as pl
from jax.experimental.pallas import tpu as pltpu
from jax.experimental.pallas import tpu_sc as plsc

_TOPK = 2
_BT = 64
_NJ = 2
_BTA = 512
_NW = 32
_CH = 64

def _sc_mesh():
    return plsc.VectorSubcoreMesh(core_axis_name="c", subcore_axis_name="s",
                                  num_cores=2, num_subcores=16)


def _router_body(x_ref, rw_ref, eidx_ref, wts_ref, cnt_ref):
    E = rw_ref.shape[0]
    l = jax.lax.dot_general(rw_ref[...], x_ref[...], (((1,), (1,)), ((), ())),
                            preferred_element_type=jnp.float32)
    ioe = lax.broadcasted_iota(jnp.int32, l.shape, 0)
    m1 = jnp.max(l, axis=0)
    i1 = jnp.min(jnp.where(l == m1[None, :], ioe, E), axis=0)
    lm = jnp.where(ioe == i1[None, :], jnp.float32(-1e30), l)
    m2 = jnp.max(lm, axis=0)
    i2 = jnp.min(jnp.where(lm == m2[None, :], ioe, E), axis=0)
    w1 = 1.0 / (1.0 + jnp.exp(m2 - m1))
    eidx_ref[0, :] = i1
    eidx_ref[1, :] = i2
    wts_ref[0, :] = w1
    wts_ref[1, :] = 1.0 - w1
    oh = ((ioe == i1[None, :]).astype(jnp.int32)
          + (ioe == i2[None, :]).astype(jnp.int32))
    for g in range(_BTA // _CH):
        cnt_ref[g, :] = jnp.sum(oh[:, g * _CH:(g + 1) * _CH], axis=1)


def _router(x, router_w):
    T, H = x.shape
    E = router_w.shape[0]
    nb = T // _BTA
    return pl.pallas_call(
        _router_body,
        grid=(nb,),
        in_specs=[
            pl.BlockSpec((_BTA, H), lambda i: (i, 0)),
            pl.BlockSpec((E, H), lambda i: (0, 0)),
        ],
        out_specs=[
            pl.BlockSpec((_TOPK, _BTA), lambda i: (0, i)),
            pl.BlockSpec((_TOPK, _BTA), lambda i: (0, i)),
            pl.BlockSpec((_BTA // _CH, E), lambda i: (i, 0)),
        ],
        out_shape=[
            jax.ShapeDtypeStruct((_TOPK, T), jnp.int32),
            jax.ShapeDtypeStruct((_TOPK, T), jnp.float32),
            jax.ShapeDtypeStruct((T // _CH, E), jnp.int32),
        ],
    )(x, router_w)


def _pos_body(eidx_ref, cnt_ref, pos_ref, te_ref, carry_ref, *, E, NB, NT):
    B = eidx_ref.shape[1]
    i = pl.program_id(0)

    @pl.when(i == 0)
    def _():
        carry_ref[...] = jnp.zeros_like(carry_ref)

    e1 = eidx_ref[0, :]
    e2 = eidx_ref[1, :]
    ioe = lax.broadcasted_iota(jnp.int32, (E, B), 0)
    oh1 = (ioe == e1[None, :]).astype(jnp.float32)
    oh2 = (ioe == e2[None, :]).astype(jnp.float32)
    ohm = oh1 + oh2

    U = (lax.broadcasted_iota(jnp.int32, (B, B), 0)
         < lax.broadcasted_iota(jnp.int32, (B, B), 1)).astype(jnp.float32)
    carry = carry_ref[:, 0:1]
    prefix = jax.lax.dot_general(ohm, U, (((1,), (0,)), ((), ())),
                                 preferred_element_type=jnp.float32) + carry

    g = jnp.sum(cnt_ref[...].astype(jnp.float32), axis=0)
    gi = g.astype(jnp.int32)
    aligned = (((gi + (_BT - 1)) >> 6) << 6).astype(jnp.float32)
    UE = (lax.broadcasted_iota(jnp.int32, (E, E), 0)
          < lax.broadcasted_iota(jnp.int32, (E, E), 1)).astype(jnp.float32)
    excl = jax.lax.dot_general(aligned[None, :], UE, (((1,), (0,)), ((), ())),
                               preferred_element_type=jnp.float32)
    excl_col = excl.reshape(E, 1)

    posmat = prefix + excl_col
    pos_ref[0, :] = jnp.sum(oh1 * posmat, axis=0).astype(jnp.int32)
    pos_ref[1, :] = jnp.sum(oh2 * posmat, axis=0).astype(jnp.int32)
    carry_ref[:, 0:1] = carry + jnp.sum(ohm, axis=1, keepdims=True)

    @pl.when(i == NB - 1)
    def _():
        tstart = excl_col.astype(jnp.int32) >> 6
        iot = lax.broadcasted_iota(jnp.int32, (E, NT), 1)
        tev = jnp.sum((tstart <= iot).astype(jnp.int32), axis=0) - 1
        te_ref[0, :] = tev


def _positions(eidx, cnts, NT):
    K, T = eidx.shape
    E = cnts.shape[1]
    NB = T // _BTA
    return pl.pallas_call(
        functools.partial(_pos_body, E=E, NB=NB, NT=NT),
        grid=(NB,),
        in_specs=[
            pl.BlockSpec((K, _BTA), lambda i: (0, i)),
            pl.BlockSpec(cnts.shape, lambda i: (0, 0)),
        ],
        out_specs=[
            pl.BlockSpec((K, _BTA), lambda i: (0, i)),
            pl.BlockSpec((1, NT), lambda i: (0, 0)),
        ],
        out_shape=[
            jax.ShapeDtypeStruct((K, T), jnp.int32),
            jax.ShapeDtypeStruct((1, NT), jnp.int32),
        ],
        scratch_shapes=[pltpu.VMEM((E, 128), jnp.float32)],
    )(eidx, cnts)


def _dispatch(x, pos, NTOT):
    T, H = x.shape

    @functools.partial(
        pl.kernel,
        out_type=jax.ShapeDtypeStruct((NTOT, H), jnp.float32),
        mesh=_sc_mesh(),
        scratch_types=[
            pltpu.VMEM((_CH,), jnp.int32),
            pltpu.VMEM((_CH,), jnp.int32),
            pltpu.VMEM((_CH, H), jnp.float32),
            pltpu.SemaphoreType.DMA,
        ],
    )
    def body(x_hbm, pos_hbm, xs_hbm, ib1, ib2, xbuf, sem):
        wid = lax.axis_index("s") * 2 + lax.axis_index("c")
        base = wid * _CH
        pltpu.sync_copy(pos_hbm.at[0, pl.ds(base, _CH)], ib1)
        pltpu.sync_copy(pos_hbm.at[1, pl.ds(base, _CH)], ib2)
        pltpu.sync_copy(x_hbm.at[pl.ds(base, _CH), :], xbuf)
        pltpu.async_copy(xbuf, xs_hbm.at[ib1], sem).wait()
        pltpu.async_copy(xbuf, xs_hbm.at[ib2], sem).wait()

    return body(x, pos)


def _c_body(te_ref, x_ref, wgx_ref, wgg_ref, wu_ref, wd_ref, y_ref, acc_ref):
    j = pl.program_id(0)
    i = pl.program_id(1)
    rows = pl.ds(i * _BT, _BT)
    x = x_ref[...]
    xh = jax.lax.dot_general(x, wgx_ref[0, 0], (((1,), (1,)), ((), ())),
                             preferred_element_type=jnp.float32)
    gt = jax.lax.dot_general(x, wgg_ref[0, 0], (((1,), (1,)), ((), ())),
                             preferred_element_type=jnp.float32)
    up = jax.lax.dot_general(x, wu_ref[0, 0], (((1,), (1,)), ((), ())),
                             preferred_element_type=jnp.float32)
    a = (gt * jax.nn.sigmoid(gt)) * xh * up
    part = jax.lax.dot_general(a, wd_ref[0], (((1,), (1,)), ((), ())),
                               preferred_element_type=jnp.float32)
    part = jnp.where(j == 0, part, acc_ref[rows, :] + part)
    acc_ref[rows, :] = part
    y_ref[...] = part


def _grouped_mlp(x_sorted, tile_expert, Wg, Wu, Wd):
    E, I2, H = Wg.shape
    I = I2 // 2
    NTOT = x_sorted.shape[0]
    NT = NTOT // _BT
    BI = I // _NJ
    WgB = Wg.reshape(E, 2 * _NJ, BI, H)

    y = pl.pallas_call(
        _c_body,
        grid_spec=pltpu.PrefetchScalarGridSpec(
            num_scalar_prefetch=1,
            grid=(_NJ, NT),
            in_specs=[
                pl.BlockSpec((_BT, H), lambda j, i, te: (i, 0)),
                pl.BlockSpec((1, 1, BI, H), lambda j, i, te: (te[i], j, 0, 0)),
                pl.BlockSpec((1, 1, BI, H), lambda j, i, te: (te[i], _NJ + j, 0, 0)),
                pl.BlockSpec((1, 1, BI, H), lambda j, i, te: (te[i], j, 0, 0)),
                pl.BlockSpec((1, H, BI), lambda j, i, te: (te[i], 0, j)),
            ],
            out_specs=pl.BlockSpec(
                (_BT, H), lambda j, i, te: (jnp.where(j == _NJ - 1, i, NT), 0)),
            scratch_shapes=[pltpu.VMEM((NTOT, H), jnp.float32)],
        ),
        out_shape=jax.ShapeDtypeStruct(((NT + 1) * _BT, H), jnp.float32),
    )(tile_expert, x_sorted, WgB, WgB, Wu.reshape(E, _NJ, BI, H), Wd)
    return y


def _combine(y, pos, wts):
    NTOT, H = y.shape
    T = pos.shape[1]
    SUB = 32
    NCH = _CH // SUB

    @functools.partial(
        pl.kernel,
        out_type=jax.ShapeDtypeStruct((T, H), jnp.float32),
        mesh=_sc_mesh(),
        scratch_types=[
            pltpu.VMEM((SUB,), jnp.int32),
            pltpu.VMEM((SUB,), jnp.int32),
            pltpu.VMEM((SUB,), jnp.float32),
            pltpu.VMEM((SUB,), jnp.float32),
            pltpu.VMEM((SUB, H), jnp.float32),
            pltpu.VMEM((SUB, H), jnp.float32),
            pltpu.VMEM((SUB, H), jnp.float32),
            pltpu.SemaphoreType.DMA,
        ],
    )
    def body(y_hbm, pos_hbm, wts_hbm, out_hbm,
             p1b, p2b, w1b, w2b, ya, yb, ob, sem):
        wid = lax.axis_index("s") * 2 + lax.axis_index("c")
        for ch in range(NCH):
            cbase = wid * _CH + ch * SUB
            pltpu.sync_copy(pos_hbm.at[0, pl.ds(cbase, SUB)], p1b)
            pltpu.sync_copy(pos_hbm.at[1, pl.ds(cbase, SUB)], p2b)
            pltpu.sync_copy(wts_hbm.at[0, pl.ds(cbase, SUB)], w1b)
            pltpu.sync_copy(wts_hbm.at[1, pl.ds(cbase, SUB)], w2b)
            pltpu.async_copy(y_hbm.at[p1b], ya, sem).wait()
            pltpu.async_copy(y_hbm.at[p2b], yb, sem).wait()
            for cv in range(SUB // 16):
                w1v = w1b[pl.ds(cv * 16, 16)]
                w2v = w2b[pl.ds(cv * 16, 16)]
                for tt in range(16):
                    t = cv * 16 + tt
                    w1 = w1v[tt]
                    w2 = w2v[tt]
                    def cloop(c, _, t=t, w1=w1, w2=w2):
                        s = pl.ds(c * 16, 16)
                        ob[t, s] = w1 * ya[t, s] + w2 * yb[t, s]
                        return 0
                    lax.fori_loop(0, H // 16, cloop, 0)
            pltpu.sync_copy(ob, out_hbm.at[pl.ds(cbase, SUB), :])

    return body(y, pos, wts)


def kernel(hidden_states, router_w, Wg, Wu, Wd):
    B, S, H = hidden_states.shape
    E = router_w.shape[0]
    T = B * S
    x = hidden_states.reshape(T, H)
    NTOT = T * _TOPK + E * _BT
    NT = NTOT // _BT

    eidx, wts, cnts = _router(x, router_w)
    pos, te = _positions(eidx, cnts, NT)
    xs = _dispatch(x, pos, NTOT)
    y = _grouped_mlp(xs, te.reshape(NT), Wg, Wu, Wd)
    out = _combine(y, pos, wts)
    return out.reshape(B, S, H)
